# baseline (speedup 1.0000x reference)
import functools

import jax
import jax.numpy as jnp
from jax import lax
from jax.experimental import pallas as pl
from jax.experimental.pallas import tpu as pltpu
from jax.experimental.pallas import tpu_sc as plsc

B = 4096
L = 200
EMB = 64
NCLS = 100
VOCAB = 1000000

NC, NS = 2, 16
NW = NC * NS
RPW = B // NW
C0 = 128
C1 = L - C0
HW = EMB // 2

VBH = 8192
NBLK = -(-VOCAB // (4 * VBH))
LASTB = (VOCAB - 1) // VBH
VPADR = NBLK * 4 * VBH


def _tc_pack(tabT):
    def pack_kernel(a_ref, b_ref, c_ref, d_ref, o_ref):
        for q, r in enumerate((a_ref, b_ref, c_ref, d_ref)):
            t = r[...].T
            lo = t[:, 0:HW].astype(jnp.bfloat16)
            hi = t[:, HW:EMB].astype(jnp.bfloat16)
            lu = jax.lax.bitcast_convert_type(lo, jnp.uint16).astype(jnp.uint32)
            hu = jax.lax.bitcast_convert_type(hi, jnp.uint16).astype(jnp.uint32)
            o_ref[:, HW * q:HW * (q + 1)] = jax.lax.bitcast_convert_type(
                lu | (hu << 16), jnp.float32)

    def _in_spec(q):
        return pl.BlockSpec((EMB, VBH),
                            lambda i, q=q: (0, jnp.minimum(4 * i + q, LASTB)))

    return pl.pallas_call(
        pack_kernel,
        grid=(NBLK,),
        in_specs=[_in_spec(0), _in_spec(1), _in_spec(2), _in_spec(3)],
        out_specs=pl.BlockSpec((VBH, 4 * HW), lambda i: (i, 0)),
        out_shape=jax.ShapeDtypeStruct((NBLK * VBH, 4 * HW), jnp.float32),
    )(tabT, tabT, tabT, tabT)


def _sc_pool_sum(x, tab):
    mesh = plsc.VectorSubcoreMesh(core_axis_name="c", subcore_axis_name="s")

    @functools.partial(
        pl.kernel,
        out_type=jax.ShapeDtypeStruct((B, EMB), jnp.float32),
        mesh=mesh,
        compiler_params=pltpu.CompilerParams(use_tc_tiling_on_sc=False,
                                             needs_layout_passes=False),
        scratch_types=[
            pltpu.VMEM((RPW, L), jnp.int32),
            pltpu.VMEM((L, HW), jnp.float32),
            pltpu.VMEM((L, HW), jnp.float32),
            pltpu.VMEM((RPW, EMB), jnp.float32),
            pltpu.SemaphoreType.DMA,
            pltpu.SemaphoreType.DMA,
        ],
    )
    def pool_kernel(x_hbm, tab_hbm, out_hbm, idx_v, rows0, rows1, acc_v, sem0, sem1):
        wid = lax.axis_index("s") * NC + lax.axis_index("c")
        base = wid * RPW
        pltpu.sync_copy(x_hbm.at[pl.ds(base, RPW), :], idx_v)

        def issue(r, rows_v, sem):
            pltpu.async_copy(
                tab_hbm.at[idx_v.at[r, pl.ds(0, C0)]], rows_v.at[pl.ds(0, C0), :], sem)
            pltpu.async_copy(
                tab_hbm.at[idx_v.at[r, pl.ds(C0, C1)]], rows_v.at[pl.ds(C0, C1), :], sem)

        def drain(rows_v, sem):
            pltpu.make_async_copy(tab_hbm.at[pl.ds(0, L), :], rows_v, sem).wait()

        mask_hi = jnp.full((16,), 0xFFFF0000, dtype=jnp.uint32)

        def accum(r, rows_v):
            zero = jnp.zeros((16,), jnp.float32)

            def body(j, accs):
                a0, a1, a2, a3 = accs
                u0 = plsc.bitcast(rows_v[j, pl.ds(0, 16)], jnp.uint32)
                u1 = plsc.bitcast(rows_v[j, pl.ds(16, 16)], jnp.uint32)
                a0 = a0 + plsc.bitcast(u0 << 16, jnp.float32)
                a1 = a1 + plsc.bitcast(u1 << 16, jnp.float32)
                a2 = a2 + plsc.bitcast(u0 & mask_hi, jnp.float32)
                a3 = a3 + plsc.bitcast(u1 & mask_hi, jnp.float32)
                return (a0, a1, a2, a3)

            accs = lax.fori_loop(0, L, body, (zero,) * 4)
            for q in range(4):
                acc_v[r, pl.ds(16 * q, 16)] = accs[q]

        issue(0, rows0, sem0)

        def outer(t, carry):
            r = 2 * t
            issue(r + 1, rows1, sem1)
            drain(rows0, sem0)
            accum(r, rows0)

            @pl.when(r + 2 < RPW)
            def _():
                issue(r + 2, rows0, sem0)

            drain(rows1, sem1)
            accum(r + 1, rows1)
            return carry

        lax.fori_loop(0, RPW // 2, outer, 0)
        pltpu.sync_copy(acc_v, out_hbm.at[pl.ds(base, RPW), :])

    return pool_kernel(x, tab)


def _tc_head(pooled_sum, Wt, b2):
    def head_kernel(p_ref, w_ref, b_ref, o_ref):
        o_ref[...] = (
            jnp.dot(p_ref[...], w_ref[...], preferred_element_type=jnp.float32)
            * (1.0 / L)
            + b_ref[...]
        )

    return pl.pallas_call(
        head_kernel,
        out_shape=jax.ShapeDtypeStruct((B, NCLS), jnp.float32),
    )(pooled_sum, Wt, b2)


def kernel(x, emb_table, W, b):
    x = x.astype(jnp.int32)
    xr = (x & ~(4 * VBH - 1)) + 4 * (x & (VBH - 1)) + ((x // VBH) & 3)
    tab2 = _tc_pack(emb_table.T)
    tab = tab2.reshape(VPADR, HW)
    pooled_sum = _sc_pool_sum(xr, tab)
    return _tc_head(pooled_sum, W.T, b.reshape(1, NCLS))


# speedup vs baseline: 1.1061x; 1.1061x over previous
import functools

import jax
import jax.numpy as jnp
from jax import lax
from jax.experimental import pallas as pl
from jax.experimental.pallas import tpu as pltpu
from jax.experimental.pallas import tpu_sc as plsc

B = 4096
L = 200
EMB = 64
NCLS = 100
VOCAB = 1000000

NC, NS = 2, 16
NW = NC * NS
RPW = B // NW
C0 = 128
C1 = L - C0
HW = EMB // 2

VBH = 8192
NBLK = -(-VOCAB // (4 * VBH))
LASTB = (VOCAB - 1) // VBH
VPADR = NBLK * 4 * VBH


def _tc_pack(tabT):
    def pack_kernel(a_ref, b_ref, c_ref, d_ref, o_ref):
        for q, r in enumerate((a_ref, b_ref, c_ref, d_ref)):
            t = r[...].astype(jnp.bfloat16).T
            lu = jax.lax.bitcast_convert_type(t[:, 0:HW], jnp.uint16).astype(jnp.uint32)
            hu = jax.lax.bitcast_convert_type(t[:, HW:EMB], jnp.uint16).astype(jnp.uint32)
            o_ref[:, HW * q:HW * (q + 1)] = jax.lax.bitcast_convert_type(
                lu | (hu << 16), jnp.float32)

    def _in_spec(q):
        return pl.BlockSpec((EMB, VBH),
                            lambda i, q=q: (0, jnp.minimum(4 * i + q, LASTB)))

    return pl.pallas_call(
        pack_kernel,
        grid=(NBLK,),
        in_specs=[_in_spec(0), _in_spec(1), _in_spec(2), _in_spec(3)],
        out_specs=pl.BlockSpec((VBH, 4 * HW), lambda i: (i, 0)),
        out_shape=jax.ShapeDtypeStruct((NBLK * VBH, 4 * HW), jnp.float32),
    )(tabT, tabT, tabT, tabT)


def _sc_pool_sum(x, tab):
    mesh = plsc.VectorSubcoreMesh(core_axis_name="c", subcore_axis_name="s")

    @functools.partial(
        pl.kernel,
        out_type=jax.ShapeDtypeStruct((B, EMB), jnp.float32),
        mesh=mesh,
        compiler_params=pltpu.CompilerParams(use_tc_tiling_on_sc=False,
                                             needs_layout_passes=False),
        scratch_types=[
            pltpu.VMEM((RPW, L), jnp.int32),
            pltpu.VMEM((L, HW), jnp.float32),
            pltpu.VMEM((L, HW), jnp.float32),
            pltpu.VMEM((RPW, EMB), jnp.float32),
            pltpu.SemaphoreType.DMA,
            pltpu.SemaphoreType.DMA,
        ],
    )
    def pool_kernel(x_hbm, tab_hbm, out_hbm, idx_v, rows0, rows1, acc_v, sem0, sem1):
        wid = lax.axis_index("s") * NC + lax.axis_index("c")
        base = wid * RPW
        pltpu.sync_copy(x_hbm.at[pl.ds(base, RPW), :], idx_v)

        def issue(r, rows_v, sem):
            pltpu.async_copy(
                tab_hbm.at[idx_v.at[r, pl.ds(0, C0)]], rows_v.at[pl.ds(0, C0), :], sem)
            pltpu.async_copy(
                tab_hbm.at[idx_v.at[r, pl.ds(C0, C1)]], rows_v.at[pl.ds(C0, C1), :], sem)

        def drain(rows_v, sem):
            pltpu.make_async_copy(tab_hbm.at[pl.ds(0, L), :], rows_v, sem).wait()

        mask_hi = jnp.full((16,), 0xFFFF0000, dtype=jnp.uint32)

        def accum(r, rows_v):
            zero = jnp.zeros((16,), jnp.float32)

            def one(j, accs):
                a0, a1, a2, a3 = accs
                u0 = plsc.bitcast(rows_v[j, pl.ds(0, 16)], jnp.uint32)
                u1 = plsc.bitcast(rows_v[j, pl.ds(16, 16)], jnp.uint32)
                a0 = a0 + plsc.bitcast(u0 << 16, jnp.float32)
                a1 = a1 + plsc.bitcast(u1 << 16, jnp.float32)
                a2 = a2 + plsc.bitcast(u0 & mask_hi, jnp.float32)
                a3 = a3 + plsc.bitcast(u1 & mask_hi, jnp.float32)
                return (a0, a1, a2, a3)

            def body(jj, accs):
                ae, ao = accs
                return (one(2 * jj, ae), one(2 * jj + 1, ao))

            ae, ao = lax.fori_loop(0, L // 2, body, ((zero,) * 4,) * 2)
            for q in range(4):
                acc_v[r, pl.ds(16 * q, 16)] = ae[q] + ao[q]

        issue(0, rows0, sem0)

        def outer(t, carry):
            r = 2 * t
            issue(r + 1, rows1, sem1)
            drain(rows0, sem0)
            accum(r, rows0)

            @pl.when(r + 2 < RPW)
            def _():
                issue(r + 2, rows0, sem0)

            drain(rows1, sem1)
            accum(r + 1, rows1)
            return carry

        lax.fori_loop(0, RPW // 2, outer, 0)
        pltpu.sync_copy(acc_v, out_hbm.at[pl.ds(base, RPW), :])

    return pool_kernel(x, tab)


def _tc_head(pooled_sum, Wt, b2):
    def head_kernel(p_ref, w_ref, b_ref, o_ref):
        o_ref[...] = (
            jnp.dot(p_ref[...], w_ref[...], preferred_element_type=jnp.float32)
            * (1.0 / L)
            + b_ref[...]
        )

    return pl.pallas_call(
        head_kernel,
        out_shape=jax.ShapeDtypeStruct((B, NCLS), jnp.float32),
    )(pooled_sum, Wt, b2)


def kernel(x, emb_table, W, b):
    x = x.astype(jnp.int32)
    xr = (x & ~(4 * VBH - 1)) + 4 * (x & (VBH - 1)) + ((x // VBH) & 3)
    tab2 = _tc_pack(emb_table.T)
    tab = tab2.reshape(VPADR, HW)
    pooled_sum = _sc_pool_sum(xr, tab)
    return _tc_head(pooled_sum, W.T, b.reshape(1, NCLS))


# speedup vs baseline: 1.2151x; 1.0985x over previous
import functools

import jax
import jax.numpy as jnp
from jax import lax
from jax.experimental import pallas as pl
from jax.experimental.pallas import tpu as pltpu
from jax.experimental.pallas import tpu_sc as plsc

B = 4096
L = 200
EMB = 64
NCLS = 100
VOCAB = 1000000

NC, NS = 2, 16
NW = NC * NS
RPW = B // NW
C0 = 128
C1 = L - C0
HW = EMB // 2

VBH = 8192
NBLK = -(-VOCAB // (4 * VBH))
LASTB = (VOCAB - 1) // VBH
VPADR = NBLK * 4 * VBH


def _tc_pack(tabT):
    def pack_kernel(a_ref, b_ref, c_ref, d_ref, o_ref):
        for q, r in enumerate((a_ref, b_ref, c_ref, d_ref)):
            t = r[...].astype(jnp.bfloat16).T
            lu = jax.lax.bitcast_convert_type(t[:, 0:HW], jnp.uint16).astype(jnp.uint32)
            hu = jax.lax.bitcast_convert_type(t[:, HW:EMB], jnp.uint16).astype(jnp.uint32)
            o_ref[:, HW * q:HW * (q + 1)] = jax.lax.bitcast_convert_type(
                lu | (hu << 16), jnp.float32)

    def _in_spec(q):
        return pl.BlockSpec((EMB, VBH),
                            lambda i, q=q: (0, jnp.minimum(4 * i + q, LASTB)))

    return pl.pallas_call(
        pack_kernel,
        grid=(NBLK,),
        in_specs=[_in_spec(0), _in_spec(1), _in_spec(2), _in_spec(3)],
        out_specs=pl.BlockSpec((VBH, 4 * HW), lambda i: (i, 0)),
        out_shape=jax.ShapeDtypeStruct((NBLK * VBH, 4 * HW), jnp.float32),
    )(tabT, tabT, tabT, tabT)


def _sc_pool_sum(x, tab):
    mesh = plsc.VectorSubcoreMesh(core_axis_name="c", subcore_axis_name="s")

    @functools.partial(
        pl.kernel,
        out_type=jax.ShapeDtypeStruct((B, EMB), jnp.float32),
        mesh=mesh,
        compiler_params=pltpu.CompilerParams(use_tc_tiling_on_sc=False,
                                             needs_layout_passes=False),
        scratch_types=[
            pltpu.VMEM((RPW, L), jnp.int32),
            pltpu.VMEM((L, HW), jnp.float32),
            pltpu.VMEM((L, HW), jnp.float32),
            pltpu.VMEM((L, HW), jnp.float32),
            pltpu.VMEM((L, HW), jnp.float32),
            pltpu.VMEM((RPW, EMB), jnp.float32),
            pltpu.SemaphoreType.DMA,
            pltpu.SemaphoreType.DMA,
            pltpu.SemaphoreType.DMA,
            pltpu.SemaphoreType.DMA,
        ],
    )
    def pool_kernel(x_hbm, tab_hbm, out_hbm, idx_v, rows0, rows1, rows2, rows3,
                    acc_v, sem0, sem1, sem2, sem3):
        wid = lax.axis_index("s") * NC + lax.axis_index("c")
        base = wid * RPW
        pltpu.sync_copy(x_hbm.at[pl.ds(base, RPW), :], idx_v)

        def issue(r, rows_v, sem):
            pltpu.async_copy(
                tab_hbm.at[idx_v.at[r, pl.ds(0, C0)]], rows_v.at[pl.ds(0, C0), :], sem)
            pltpu.async_copy(
                tab_hbm.at[idx_v.at[r, pl.ds(C0, C1)]], rows_v.at[pl.ds(C0, C1), :], sem)

        def drain(rows_v, sem):
            pltpu.make_async_copy(tab_hbm.at[pl.ds(0, L), :], rows_v, sem).wait()

        mask_hi = jnp.full((16,), 0xFFFF0000, dtype=jnp.uint32)

        def accum(r, rows_v):
            zero = jnp.zeros((16,), jnp.float32)

            def one(j, accs):
                a0, a1, a2, a3 = accs
                u0 = plsc.bitcast(rows_v[j, pl.ds(0, 16)], jnp.uint32)
                u1 = plsc.bitcast(rows_v[j, pl.ds(16, 16)], jnp.uint32)
                a0 = a0 + plsc.bitcast(u0 << 16, jnp.float32)
                a1 = a1 + plsc.bitcast(u1 << 16, jnp.float32)
                a2 = a2 + plsc.bitcast(u0 & mask_hi, jnp.float32)
                a3 = a3 + plsc.bitcast(u1 & mask_hi, jnp.float32)
                return (a0, a1, a2, a3)

            def body(jj, accs):
                ae, ao = accs
                return (one(2 * jj, ae), one(2 * jj + 1, ao))

            ae, ao = lax.fori_loop(0, L // 2, body, ((zero,) * 4,) * 2)
            for q in range(4):
                acc_v[r, pl.ds(16 * q, 16)] = ae[q] + ao[q]

        bufs = (rows0, rows1, rows2, rows3)
        sems = (sem0, sem1, sem2, sem3)
        ND = 4
        for k in range(ND - 1):
            issue(k, bufs[k], sems[k])

        def outer(t, carry):
            r = ND * t
            for k in range(ND):
                rr = r + k
                nk = (k + ND - 1) % ND

                @pl.when(rr + ND - 1 < RPW)
                def _(rr=rr, nk=nk):
                    issue(rr + ND - 1, bufs[nk], sems[nk])

                drain(bufs[k], sems[k])
                accum(rr, bufs[k])
            return carry

        lax.fori_loop(0, RPW // ND, outer, 0)
        pltpu.sync_copy(acc_v, out_hbm.at[pl.ds(base, RPW), :])

    return pool_kernel(x, tab)


def _tc_head(pooled_sum, Wt, b2):
    def head_kernel(p_ref, w_ref, b_ref, o_ref):
        o_ref[...] = (
            jnp.dot(p_ref[...], w_ref[...], preferred_element_type=jnp.float32)
            * (1.0 / L)
            + b_ref[...]
        )

    return pl.pallas_call(
        head_kernel,
        out_shape=jax.ShapeDtypeStruct((B, NCLS), jnp.float32),
    )(pooled_sum, Wt, b2)


def kernel(x, emb_table, W, b):
    x = x.astype(jnp.int32)
    xr = (x & ~(4 * VBH - 1)) + 4 * (x & (VBH - 1)) + ((x // VBH) & 3)
    tab2 = _tc_pack(emb_table.T)
    tab = tab2.reshape(VPADR, HW)
    pooled_sum = _sc_pool_sum(xr, tab)
    return _tc_head(pooled_sum, W.T, b.reshape(1, NCLS))


# speedup vs baseline: 1.4987x; 1.2334x over previous
import functools

import jax
import jax.numpy as jnp
from jax import lax
from jax.experimental import pallas as pl
from jax.experimental.pallas import tpu as pltpu
from jax.experimental.pallas import tpu_sc as plsc

B = 4096
L = 200
EMB = 64
NCLS = 100
VOCAB = 1000000

NC, NS = 2, 16
NW = NC * NS
RPW = B // NW
C0 = 128
C1 = L - C0
HW = EMB // 2

VBH = 12288
NBLK = -(-VOCAB // (4 * VBH))
LASTB = (VOCAB - 1) // VBH
VPADR = NBLK * 4 * VBH


def _tc_pack(tabT):
    def pack_kernel(a_ref, b_ref, c_ref, d_ref, o_ref):
        for q, r in enumerate((a_ref, b_ref, c_ref, d_ref)):
            t = r[...].astype(jnp.bfloat16).T
            lu = jax.lax.bitcast_convert_type(t[:, 0:HW], jnp.uint16).astype(jnp.uint32)
            hu = jax.lax.bitcast_convert_type(t[:, HW:EMB], jnp.uint16).astype(jnp.uint32)
            o_ref[:, HW * q:HW * (q + 1)] = jax.lax.bitcast_convert_type(
                lu | (hu << 16), jnp.float32)

    def _in_spec(q):
        return pl.BlockSpec((EMB, VBH),
                            lambda i, q=q: (0, jnp.minimum(4 * i + q, LASTB)))

    return pl.pallas_call(
        pack_kernel,
        grid=(NBLK,),
        in_specs=[_in_spec(0), _in_spec(1), _in_spec(2), _in_spec(3)],
        out_specs=pl.BlockSpec((VBH, 4 * HW), lambda i: (i, 0)),
        out_shape=jax.ShapeDtypeStruct((NBLK * VBH, 4 * HW), jnp.float32),
    )(tabT, tabT, tabT, tabT)


def _sc_pool_sum(x, tab):
    mesh = plsc.VectorSubcoreMesh(core_axis_name="c", subcore_axis_name="s")

    @functools.partial(
        pl.kernel,
        out_type=jax.ShapeDtypeStruct((B, EMB), jnp.float32),
        mesh=mesh,
        compiler_params=pltpu.CompilerParams(use_tc_tiling_on_sc=False,
                                             needs_layout_passes=False),
        scratch_types=[
            pltpu.VMEM((RPW, L), jnp.int32),
            pltpu.VMEM((L, HW), jnp.float32),
            pltpu.VMEM((L, HW), jnp.float32),
            pltpu.VMEM((L, HW), jnp.float32),
            pltpu.VMEM((L, HW), jnp.float32),
            pltpu.VMEM((RPW, EMB), jnp.float32),
            pltpu.SemaphoreType.DMA,
            pltpu.SemaphoreType.DMA,
            pltpu.SemaphoreType.DMA,
            pltpu.SemaphoreType.DMA,
        ],
    )
    def pool_kernel(x_hbm, tab_hbm, out_hbm, idx_v, rows0, rows1, rows2, rows3,
                    acc_v, sem0, sem1, sem2, sem3):
        wid = lax.axis_index("s") * NC + lax.axis_index("c")
        base = wid * RPW
        pltpu.sync_copy(x_hbm.at[pl.ds(base, RPW), :], idx_v)

        def issue(r, rows_v, sem):
            pltpu.async_copy(
                tab_hbm.at[idx_v.at[r, pl.ds(0, C0)]], rows_v.at[pl.ds(0, C0), :], sem)
            pltpu.async_copy(
                tab_hbm.at[idx_v.at[r, pl.ds(C0, C1)]], rows_v.at[pl.ds(C0, C1), :], sem)

        def drain(rows_v, sem):
            pltpu.make_async_copy(tab_hbm.at[pl.ds(0, L), :], rows_v, sem).wait()

        mask_hi = jnp.full((16,), 0xFFFF0000, dtype=jnp.uint32)

        def accum(r, rows_v):
            zero = jnp.zeros((16,), jnp.float32)

            def one(j, accs):
                a0, a1, a2, a3 = accs
                u0 = plsc.bitcast(rows_v[j, pl.ds(0, 16)], jnp.uint32)
                u1 = plsc.bitcast(rows_v[j, pl.ds(16, 16)], jnp.uint32)
                a0 = a0 + plsc.bitcast(u0 << 16, jnp.float32)
                a1 = a1 + plsc.bitcast(u1 << 16, jnp.float32)
                a2 = a2 + plsc.bitcast(u0 & mask_hi, jnp.float32)
                a3 = a3 + plsc.bitcast(u1 & mask_hi, jnp.float32)
                return (a0, a1, a2, a3)

            def body(jj, accs):
                ae, ao = accs
                return (one(2 * jj, ae), one(2 * jj + 1, ao))

            ae, ao = lax.fori_loop(0, L // 2, body, ((zero,) * 4,) * 2)
            for q in range(4):
                acc_v[r, pl.ds(16 * q, 16)] = ae[q] + ao[q]

        bufs = (rows0, rows1, rows2, rows3)
        sems = (sem0, sem1, sem2, sem3)
        ND = 4
        for k in range(ND - 1):
            issue(k, bufs[k], sems[k])

        def outer(t, carry):
            r = ND * t
            for k in range(ND):
                rr = r + k
                nk = (k + ND - 1) % ND

                @pl.when(rr + ND - 1 < RPW)
                def _(rr=rr, nk=nk):
                    issue(rr + ND - 1, bufs[nk], sems[nk])

                drain(bufs[k], sems[k])
                accum(rr, bufs[k])
            return carry

        lax.fori_loop(0, RPW // ND, outer, 0)
        pltpu.sync_copy(acc_v, out_hbm.at[pl.ds(base, RPW), :])

    return pool_kernel(x, tab)


def _tc_head(pooled_sum, Wt, b2):
    def head_kernel(p_ref, w_ref, b_ref, o_ref):
        o_ref[...] = (
            jnp.dot(p_ref[...], w_ref[...], preferred_element_type=jnp.float32)
            * (1.0 / L)
            + b_ref[...]
        )

    return pl.pallas_call(
        head_kernel,
        out_shape=jax.ShapeDtypeStruct((B, NCLS), jnp.float32),
    )(pooled_sum, Wt, b2)


def kernel(x, emb_table, W, b):
    x = x.astype(jnp.int32)
    xr = (x & ~(4 * VBH - 1)) + 4 * (x & (VBH - 1)) + ((x // VBH) & 3)
    tab2 = _tc_pack(emb_table.T)
    tab = tab2.reshape(VPADR, HW)
    pooled_sum = _sc_pool_sum(xr, tab)
    return _tc_head(pooled_sum, W.T, b.reshape(1, NCLS))
